# baseline (speedup 1.0000x reference)
import functools

import jax
import jax.numpy as jnp
from jax import lax
from jax.experimental import pallas as pl
from jax.experimental.pallas import tpu as pltpu
from jax.experimental.pallas import tpu_sc as plsc

_NUM_CORES = 2
_NUM_SUBCORES = 16
_NW = _NUM_CORES * _NUM_SUBCORES
_L = 16


def _make_sc_kernel(n_rows):
    rows_per_w = n_rows // _NW
    chunks = rows_per_w // _L
    mesh = plsc.VectorSubcoreMesh(
        core_axis_name="c", subcore_axis_name="s", num_cores=_NUM_CORES
    )

    @functools.partial(
        pl.kernel,
        out_type=jax.ShapeDtypeStruct((3, n_rows), jnp.float32),
        mesh=mesh,
        scratch_types=[
            pltpu.VMEM((9, rows_per_w), jnp.float32),
            pltpu.VMEM((3, rows_per_w), jnp.float32),
            pltpu.SemaphoreType.DMA,
            pltpu.SemaphoreType.DMA,
            pltpu.SemaphoreType.DMA,
            pltpu.SemaphoreType.DMA,
        ],
        compiler_params=pltpu.CompilerParams(
            needs_layout_passes=False,
        ),
    )
    def sc_kernel(x_hbm, out_hbm, x_v, out_v, ld0, ld1, st0, st1):
        wid = lax.axis_index("s") * _NUM_CORES + lax.axis_index("c")
        base = wid * rows_per_w
        half = rows_per_w // 2
        cp0 = pltpu.make_async_copy(
            x_hbm.at[:, pl.ds(base, half)], x_v.at[:, pl.ds(0, half)], ld0)
        cp1 = pltpu.make_async_copy(
            x_hbm.at[:, pl.ds(base + half, half)],
            x_v.at[:, pl.ds(half, half)], ld1)
        cp0.start()
        cp1.start()

        zero_f = jnp.zeros((_L,), jnp.float32)

        def get_m(a, b, c):
            p = (a > jnp.maximum(b, c)).astype(jnp.int32)
            q = (c > jnp.maximum(a, b)).astype(jnp.int32)
            return p - q

        def body(i):
            sl = pl.ds(i * _L, _L)
            xs = [x_v[c, sl] for c in range(9)]
            m_u = get_m(*xs[0:3])
            m_n = get_m(*xs[3:6])
            m_d = get_m(*xs[6:9])
            calc = jnp.abs(m_n) * (m_u + m_d + m_n)
            s = jnp.sign(calc)
            keep_u = s == m_u
            keep_n = s == m_n
            keep_d = s == m_d
            c0 = calc == 0
            c1 = calc == 1
            def val(g, keep):
                raw = jnp.where(c0, xs[3 * g + 1],
                                jnp.where(c1, xs[3 * g], xs[3 * g + 2]))
                return jnp.where(keep, raw, zero_f)

            val_u = val(0, keep_u)
            val_n = val(1, keep_n)
            val_d = val(2, keep_d)
            w_u = (val_u >= val_n) & (val_u >= val_d)
            w_n = jnp.logical_not(w_u) & (val_n >= val_d)
            k_win = jnp.where(w_u, keep_u, jnp.where(w_n, keep_n, keep_d))
            for j in range(3):
                sel = jnp.where(w_u, xs[j],
                                jnp.where(w_n, xs[3 + j], xs[6 + j]))
                out_v[j, sl] = jnp.where(k_win, sel, zero_f)

        hchunks = chunks // 2
        cp0.wait()
        plsc.parallel_loop(0, hchunks, 1, unroll=2)(body)
        w0 = pltpu.make_async_copy(
            out_v.at[:, pl.ds(0, half)],
            out_hbm.at[:, pl.ds(base, half)], st0)
        w0.start()
        cp1.wait()
        plsc.parallel_loop(hchunks, chunks, 1, unroll=2)(body)
        w1 = pltpu.make_async_copy(
            out_v.at[:, pl.ds(half, half)],
            out_hbm.at[:, pl.ds(base + half, half)], st1)
        w1.start()
        w0.wait()
        w1.wait()

    return sc_kernel


def kernel(inputs):
    n_rows, n_feat = inputs.shape
    assert n_feat == 9 and n_rows % (_NW * _L) == 0
    out_t = _make_sc_kernel(n_rows)(inputs.T)
    return out_t.T


# speedup vs baseline: 1.0057x; 1.0057x over previous
import functools

import jax
import jax.numpy as jnp
from jax import lax
from jax.experimental import pallas as pl
from jax.experimental.pallas import tpu as pltpu
from jax.experimental.pallas import tpu_sc as plsc

_NUM_CORES = 2
_NUM_SUBCORES = 16
_NW = _NUM_CORES * _NUM_SUBCORES
_L = 16


def _make_sc_kernel(n_rows):
    rows_per_w = n_rows // _NW
    chunks = rows_per_w // _L
    mesh = plsc.VectorSubcoreMesh(
        core_axis_name="c", subcore_axis_name="s", num_cores=_NUM_CORES
    )

    @functools.partial(
        pl.kernel,
        out_type=jax.ShapeDtypeStruct((3, n_rows), jnp.float32),
        mesh=mesh,
        scratch_types=[
            pltpu.VMEM((9, rows_per_w), jnp.float32),
            pltpu.VMEM((3, rows_per_w), jnp.float32),
        ],
        compiler_params=pltpu.CompilerParams(
            needs_layout_passes=False,
        ),
    )
    def sc_kernel(x_hbm, out_hbm, x_v, out_v):
        wid = lax.axis_index("s") * _NUM_CORES + lax.axis_index("c")
        base = wid * rows_per_w
        pltpu.sync_copy(x_hbm.at[:, pl.ds(base, rows_per_w)], x_v)

        zero_f = jnp.zeros((_L,), jnp.float32)

        def get_m(a, b, c):
            p = (a > jnp.maximum(b, c)).astype(jnp.int32)
            q = (c > jnp.maximum(a, b)).astype(jnp.int32)
            return p - q

        def body(i):
            sl = pl.ds(i * _L, _L)
            xs = [x_v[c, sl] for c in range(9)]
            m_u = get_m(*xs[0:3])
            m_n = get_m(*xs[3:6])
            m_d = get_m(*xs[6:9])
            calc = jnp.abs(m_n) * (m_u + m_d + m_n)
            s = jnp.sign(calc)
            keep_u = s == m_u
            keep_n = s == m_n
            keep_d = s == m_d
            c0 = calc == 0
            c1 = calc == 1
            def val(g, keep):
                raw = jnp.where(c0, xs[3 * g + 1],
                                jnp.where(c1, xs[3 * g], xs[3 * g + 2]))
                return jnp.where(keep, raw, zero_f)

            val_u = val(0, keep_u)
            val_n = val(1, keep_n)
            val_d = val(2, keep_d)
            w_u = (val_u >= val_n) & (val_u >= val_d)
            w_n = jnp.logical_not(w_u) & (val_n >= val_d)
            k_win = jnp.where(w_u, keep_u, jnp.where(w_n, keep_n, keep_d))
            for j in range(3):
                sel = jnp.where(w_u, xs[j],
                                jnp.where(w_n, xs[3 + j], xs[6 + j]))
                out_v[j, sl] = jnp.where(k_win, sel, zero_f)

        plsc.parallel_loop(0, chunks, 1, unroll=2)(body)
        pltpu.sync_copy(out_v, out_hbm.at[:, pl.ds(base, rows_per_w)])

    return sc_kernel


def kernel(inputs):
    n_rows, n_feat = inputs.shape
    assert n_feat == 9 and n_rows % (_NW * _L) == 0
    out_t = _make_sc_kernel(n_rows)(inputs.T)
    return out_t.T
